# baseline (speedup 1.0000x reference)
import jax
import jax.numpy as jnp
from jax import lax
from jax.experimental import pallas as pl
from jax.experimental.pallas import tpu as pltpu, tpu_sc as plsc

_NUM = 4096
_CHILD = 65536
_CPN = 16
_BATCH = 512
_NC, _NS = 2, 16
_NW = _NC * _NS
_CH_W = _CHILD // _NW
_NODES_W = _NUM // _NW
_NBLK_N = _NODES_W // _CPN
_NG = _NODES_W // 8
_RB = 16
_NBLK = _BATCH // _RB
_SROW = 2 * _NODES_W


def _vlog(x):
    ix = lax.bitcast_convert_type(x, jnp.int32)
    e = lax.shift_right_arithmetic(ix - 0x3F3504F3, 23)
    m = lax.bitcast_convert_type(ix - lax.shift_left(e, 23), jnp.float32)
    s = (m - 1.0) / (m + 1.0)
    z = s * s
    p = z * (jnp.float32(1.0 / 3.0) + z * (jnp.float32(0.2)
             + z * jnp.float32(1.0 / 7.0)))
    return e.astype(jnp.float32) * jnp.float32(0.6931471805599453) \
        + 2.0 * (s + s * p)


def _sc_body(x_hbm, lp_hbm, out_hbm, lp_v, lpt_v, lse_v, idx_t, xbuf,
             stg, st, sem0, sem1, osem0, osem1):
    wid = lax.axis_index("s") * _NC + lax.axis_index("c")
    ch0 = wid * _CH_W
    col0 = wid * _NODES_W
    bi = lax.iota(jnp.int32, 16)
    bi16 = bi * 16
    pat = (bi & 7) * 16 + lax.shift_left(lax.shift_right_logical(bi, 3), 3)
    patf = lax.shift_left(lax.shift_right_logical(bi, 3), 4) + (bi & 7)

    pltpu.sync_copy(lp_hbm.at[pl.ds(ch0, _CH_W)], lp_v)
    for k in range(_NBLK_N):
        acc = None
        for c in range(_CPN):
            g = plsc.load_gather(lp_v, [bi16 + (k * 256 + c)])
            e = jnp.exp(g)
            acc = e if acc is None else acc + e
        lse_v[pl.ds(k * _CPN, _CPN)] = _vlog(acc)
    for g in range(_NG):
        for c in range(8):
            idx = pat + (g * 128 + c)
            idx_t[g * 8 + c, :] = idx
            lpt_v[c, pl.ds(g * _CPN, _CPN)] = plsc.load_gather(lp_v, [idx])

    _HALF = _RB * _CH_W

    def _start(half, sem, blk):
        for b in range(_RB):
            pltpu.make_async_copy(
                x_hbm.at[blk * _RB + b, pl.ds(ch0, _CH_W)],
                xbuf.at[pl.ds(half * _HALF + b * _CH_W, _CH_W)], sem
            ).start()

    def _wait(sem):
        pltpu.make_async_copy(
            x_hbm.at[0, pl.ds(0, _HALF)], xbuf.at[pl.ds(0, _HALF)], sem
        ).wait()

    def _out_start(half, osem, blk):
        pltpu.make_async_copy(
            st.at[pl.ds(half * _RB, _RB), :],
            out_hbm.at[pl.ds(blk * _RB, _RB), pl.ds(col0, _NODES_W)],
            osem
        ).start()

    def _out_wait(osem):
        pltpu.make_async_copy(
            st.at[pl.ds(0, _RB), :],
            out_hbm.at[pl.ds(0, _RB), pl.ds(col0, _NODES_W)], osem
        ).wait()

    def _compute(xoff, soff):
        @plsc.parallel_loop(0, _NG, 1, unroll=1)
        def group_body(g):
            gcol = g * _CPN
            idxs = [idx_t[g * 8 + c, :] for c in range(8)]
            lpv = [lpt_v[c, pl.ds(gcol, _CPN)] for c in range(8)]
            for b in range(0, _RB, 4):
                rs = [xbuf.at[pl.ds(pl.multiple_of(xoff + (b + j) * _CH_W,
                                                   _CH_W), _CH_W)]
                      for j in range(4)]
                accs = [[None] * 2 for _ in range(4)]
                for c in range(8):
                    gs = [plsc.load_gather(rs[j], [idxs[c]])
                          for j in range(4)]
                    for j in range(4):
                        t = jnp.exp(gs[j] + lpv[c])
                        p = accs[j][c % 2]
                        accs[j][c % 2] = t if p is None else p + t
                for j in range(4):
                    stg[pl.ds((b + j) * _SROW + gcol, _CPN)] = (
                        accs[j][0] + accs[j][1])

        @plsc.parallel_loop(0, _RB, 1, unroll=1)
        def row_body(b):
            rowref = stg.at[pl.ds(pl.multiple_of(b * _SROW, _SROW), _SROW)]
            for k in range(_NBLK_N):
                iA = patf + (k * 32)
                iB = iA + 8
                A = plsc.load_gather(rowref, [iA])
                B = plsc.load_gather(rowref, [iB])
                sl = pl.ds(k * _CPN, _CPN)
                st[soff + b, sl] = _vlog(A + B) - lse_v[sl]

    _start(0, sem0, 0)

    def blk_body(i, _):
        par = lax.rem(i, 2)
        even = par == 0

        @pl.when((i < _NBLK - 1) & even)
        def _():
            _start(1, sem1, i + 1)

        @pl.when((i < _NBLK - 1) & ~even)
        def _():
            _start(0, sem0, i + 1)

        @pl.when(even)
        def _():
            _wait(sem0)

        @pl.when(~even)
        def _():
            _wait(sem1)

        @pl.when((i > 1) & even)
        def _():
            _out_wait(osem0)

        @pl.when((i > 1) & ~even)
        def _():
            _out_wait(osem1)

        _compute(pl.multiple_of(par * _HALF, _CH_W), par * _RB)

        @pl.when(even)
        def _():
            _out_start(0, osem0, i)

        @pl.when(~even)
        def _():
            _out_start(1, osem1, i)

        return 0
    lax.fori_loop(0, _NBLK, blk_body, 0)

    _out_wait(osem0)
    _out_wait(osem1)


def kernel(input, logparams):
    mesh = plsc.VectorSubcoreMesh(core_axis_name="c", subcore_axis_name="s")
    f = pl.kernel(
        _sc_body,
        out_type=jax.ShapeDtypeStruct((_BATCH, _NUM), jnp.float32),
        mesh=mesh,
        compiler_params=pltpu.CompilerParams(needs_layout_passes=False),
        scratch_types=[
            pltpu.VMEM((_CH_W,), jnp.float32),
            pltpu.VMEM((8, _NG * _CPN), jnp.float32),
            pltpu.VMEM((_NODES_W,), jnp.float32),
            pltpu.VMEM((_NODES_W, _CPN), jnp.int32),
            pltpu.VMEM((2 * _RB * _CH_W,), jnp.float32),
            pltpu.VMEM((_RB * _SROW,), jnp.float32),
            pltpu.VMEM((2 * _RB, _NODES_W), jnp.float32),
            pltpu.SemaphoreType.DMA,
            pltpu.SemaphoreType.DMA,
            pltpu.SemaphoreType.DMA,
            pltpu.SemaphoreType.DMA,
        ],
    )
    return f(input, logparams)


# speedup vs baseline: 1.2551x; 1.2551x over previous
import jax
import jax.numpy as jnp
from jax import lax
from jax.experimental import pallas as pl
from jax.experimental.pallas import tpu as pltpu, tpu_sc as plsc

_NUM = 4096
_CHILD = 65536
_CPN = 16
_BATCH = 512
_NC, _NS = 2, 16
_NW = _NC * _NS
_CH_W = _CHILD // _NW
_NODES_W = _NUM // _NW
_NBLK_N = _NODES_W // _CPN
_NG = _NODES_W // 8
_RB = 16
_NBLK = _BATCH // _RB
_SROW = 2 * _NODES_W


def _vlog(x):
    ix = lax.bitcast_convert_type(x, jnp.int32)
    e = lax.shift_right_arithmetic(ix - 0x3F3504F3, 23)
    m = lax.bitcast_convert_type(ix - lax.shift_left(e, 23), jnp.float32)
    s = (m - 1.0) / (m + 1.0)
    z = s * s
    p = z * (jnp.float32(1.0 / 3.0) + z * (jnp.float32(0.2)
             + z * jnp.float32(1.0 / 7.0)))
    return e.astype(jnp.float32) * jnp.float32(0.6931471805599453) \
        + 2.0 * (s + s * p)


def _sc_body(x_hbm, lp_hbm, out_hbm, lp_v, lpt_v, lse_v, idx_t, xbuf,
             stg, st, sem0, sem1, osem0, osem1):
    wid = lax.axis_index("s") * _NC + lax.axis_index("c")
    ch0 = wid * _CH_W
    col0 = wid * _NODES_W
    bi = lax.iota(jnp.int32, 16)
    bi16 = bi * 16
    pat = (bi & 7) * 16 + lax.shift_left(lax.shift_right_logical(bi, 3), 3)
    patf = lax.shift_left(lax.shift_right_logical(bi, 3), 4) + (bi & 7)

    pltpu.sync_copy(lp_hbm.at[pl.ds(ch0, _CH_W)], lp_v)
    for k in range(_NBLK_N):
        acc = None
        for c in range(_CPN):
            g = plsc.load_gather(lp_v, [bi16 + (k * 256 + c)])
            e = jnp.exp(g)
            acc = e if acc is None else acc + e
        lse_v[pl.ds(k * _CPN, _CPN)] = _vlog(acc)
    for g in range(_NG):
        for c in range(8):
            idx = pat + (g * 128 + c)
            idx_t[g * 8 + c, :] = idx
            lpt_v[c, pl.ds(g * _CPN, _CPN)] = plsc.load_gather(lp_v, [idx])

    _HALF = _RB * _CH_W

    def _start(half, sem, blk):
        for b in range(_RB):
            pltpu.make_async_copy(
                x_hbm.at[blk * _RB + b, pl.ds(ch0, _CH_W)],
                xbuf.at[pl.ds(half * _HALF + b * _CH_W, _CH_W)], sem
            ).start()

    def _wait(sem):
        pltpu.make_async_copy(
            x_hbm.at[0, pl.ds(0, _HALF)], xbuf.at[pl.ds(0, _HALF)], sem
        ).wait()

    def _out_start(half, osem, blk):
        pltpu.make_async_copy(
            st.at[pl.ds(half * _RB, _RB), :],
            out_hbm.at[pl.ds(blk * _RB, _RB), pl.ds(col0, _NODES_W)],
            osem
        ).start()

    def _out_wait(osem):
        pltpu.make_async_copy(
            st.at[pl.ds(0, _RB), :],
            out_hbm.at[pl.ds(0, _RB), pl.ds(col0, _NODES_W)], osem
        ).wait()

    def _compute(xoff, soff):
        @plsc.parallel_loop(0, _NG, 1, unroll=1)
        def group_body(g):
            gcol = g * _CPN
            idxs = [idx_t[g * 8 + c, :] for c in range(8)]
            lpv = [lpt_v[c, pl.ds(gcol, _CPN)] for c in range(8)]
            for b in range(0, _RB, 2):
                r0 = xbuf.at[pl.ds(pl.multiple_of(xoff + b * _CH_W,
                                                  _CH_W), _CH_W)]
                r1 = xbuf.at[pl.ds(pl.multiple_of(xoff + (b + 1) * _CH_W,
                                                  _CH_W), _CH_W)]
                a0 = [None] * 2
                a1 = [None] * 2
                for c in range(8):
                    g0 = plsc.load_gather(r0, [idxs[c]])
                    g1 = plsc.load_gather(r1, [idxs[c]])
                    t0 = jnp.exp(g0 + lpv[c])
                    t1 = jnp.exp(g1 + lpv[c])
                    p0, p1 = a0[c % 2], a1[c % 2]
                    a0[c % 2] = t0 if p0 is None else p0 + t0
                    a1[c % 2] = t1 if p1 is None else p1 + t1
                stg[pl.ds(b * _SROW + gcol, _CPN)] = a0[0] + a0[1]
                stg[pl.ds((b + 1) * _SROW + gcol, _CPN)] = a1[0] + a1[1]

        @plsc.parallel_loop(0, _RB, 2, unroll=1)
        def row_body(b):
            ro0 = pl.multiple_of(b * _SROW, _SROW)
            r0 = stg.at[pl.ds(ro0, _SROW)]
            r1 = stg.at[pl.ds(ro0 + _SROW, _SROW)]
            for k in range(_NBLK_N):
                iA = patf + (k * 32)
                iB = iA + 8
                A0 = plsc.load_gather(r0, [iA])
                B0 = plsc.load_gather(r0, [iB])
                A1 = plsc.load_gather(r1, [iA])
                B1 = plsc.load_gather(r1, [iB])
                sl = pl.ds(k * _CPN, _CPN)
                lse = lse_v[sl]
                st[soff + b, sl] = _vlog(A0 + B0) - lse
                st[soff + b + 1, sl] = _vlog(A1 + B1) - lse

    _start(0, sem0, 0)

    def blk_body(i, _):
        par = lax.rem(i, 2)
        even = par == 0

        @pl.when((i < _NBLK - 1) & even)
        def _():
            _start(1, sem1, i + 1)

        @pl.when((i < _NBLK - 1) & ~even)
        def _():
            _start(0, sem0, i + 1)

        @pl.when(even)
        def _():
            _wait(sem0)

        @pl.when(~even)
        def _():
            _wait(sem1)

        @pl.when((i > 1) & even)
        def _():
            _out_wait(osem0)

        @pl.when((i > 1) & ~even)
        def _():
            _out_wait(osem1)

        _compute(pl.multiple_of(par * _HALF, _CH_W), par * _RB)

        @pl.when(even)
        def _():
            _out_start(0, osem0, i)

        @pl.when(~even)
        def _():
            _out_start(1, osem1, i)

        return 0
    lax.fori_loop(0, _NBLK, blk_body, 0)

    _out_wait(osem0)
    _out_wait(osem1)


def kernel(input, logparams):
    mesh = plsc.VectorSubcoreMesh(core_axis_name="c", subcore_axis_name="s")
    f = pl.kernel(
        _sc_body,
        out_type=jax.ShapeDtypeStruct((_BATCH, _NUM), jnp.float32),
        mesh=mesh,
        compiler_params=pltpu.CompilerParams(needs_layout_passes=False),
        scratch_types=[
            pltpu.VMEM((_CH_W,), jnp.float32),
            pltpu.VMEM((8, _NG * _CPN), jnp.float32),
            pltpu.VMEM((_NODES_W,), jnp.float32),
            pltpu.VMEM((_NODES_W, _CPN), jnp.int32),
            pltpu.VMEM((2 * _RB * _CH_W,), jnp.float32),
            pltpu.VMEM((_RB * _SROW,), jnp.float32),
            pltpu.VMEM((2 * _RB, _NODES_W), jnp.float32),
            pltpu.SemaphoreType.DMA,
            pltpu.SemaphoreType.DMA,
            pltpu.SemaphoreType.DMA,
            pltpu.SemaphoreType.DMA,
        ],
    )
    return f(input, logparams)


# speedup vs baseline: 1.3635x; 1.0863x over previous
import jax
import jax.numpy as jnp
from jax import lax
from jax.experimental import pallas as pl
from jax.experimental.pallas import tpu as pltpu, tpu_sc as plsc

_NUM = 4096
_CHILD = 65536
_CPN = 16
_BATCH = 512
_NC, _NS = 2, 16
_NW = _NC * _NS
_CH_W = _CHILD // _NW
_NODES_W = _NUM // _NW
_NBLK_N = _NODES_W // _CPN
_NG = _NODES_W // 8
_RB = 16
_NBLK = _BATCH // _RB
_SROW = 2 * _NODES_W


def _vlog(x):
    ix = lax.bitcast_convert_type(x, jnp.int32)
    e = lax.shift_right_arithmetic(ix - 0x3F3504F3, 23)
    m = lax.bitcast_convert_type(ix - lax.shift_left(e, 23), jnp.float32)
    s = (m - 1.0) / (m + 1.0)
    z = s * s
    p = z * (jnp.float32(1.0 / 3.0) + z * (jnp.float32(0.2)
             + z * jnp.float32(1.0 / 7.0)))
    return e.astype(jnp.float32) * jnp.float32(0.6931471805599453) \
        + 2.0 * (s + s * p)


def _sc_body(x_hbm, lp_hbm, out_hbm, lp_v, lpt_v, lse_v, idx_t, xbuf,
             stg, st, sem0, sem1, osem0, osem1):
    wid = lax.axis_index("s") * _NC + lax.axis_index("c")
    ch0 = wid * _CH_W
    col0 = wid * _NODES_W
    bi = lax.iota(jnp.int32, 16)
    bi16 = bi * 16
    pat = (bi & 7) * 16 + lax.shift_left(lax.shift_right_logical(bi, 3), 3)
    patf = lax.shift_left(lax.shift_right_logical(bi, 3), 4) + (bi & 7)

    pltpu.sync_copy(lp_hbm.at[pl.ds(ch0, _CH_W)], lp_v)
    for k in range(_NBLK_N):
        acc = None
        for c in range(_CPN):
            g = plsc.load_gather(lp_v, [bi16 + (k * 256 + c)])
            e = jnp.exp(g)
            acc = e if acc is None else acc + e
        lse_v[pl.ds(k * _CPN, _CPN)] = _vlog(acc)
    for g in range(_NG):
        for c in range(8):
            idx = pat + (g * 128 + c)
            idx_t[g * 8 + c, :] = idx
            lpt_v[c, pl.ds(g * _CPN, _CPN)] = plsc.load_gather(lp_v, [idx])

    _HALF = _RB * _CH_W

    def _start(half, sem, blk):
        for b in range(_RB):
            pltpu.make_async_copy(
                x_hbm.at[blk * _RB + b, pl.ds(ch0, _CH_W)],
                xbuf.at[pl.ds(half * _HALF + b * _CH_W, _CH_W)], sem
            ).start()

    def _wait(sem):
        pltpu.make_async_copy(
            x_hbm.at[0, pl.ds(0, _HALF)], xbuf.at[pl.ds(0, _HALF)], sem
        ).wait()

    def _out_start(half, osem, blk):
        pltpu.make_async_copy(
            st.at[pl.ds(half * _RB, _RB), :],
            out_hbm.at[pl.ds(blk * _RB, _RB), pl.ds(col0, _NODES_W)],
            osem
        ).start()

    def _out_wait(osem):
        pltpu.make_async_copy(
            st.at[pl.ds(0, _RB), :],
            out_hbm.at[pl.ds(0, _RB), pl.ds(col0, _NODES_W)], osem
        ).wait()

    def _compute(xoff, soff):
        @plsc.parallel_loop(0, _NG, 1, unroll=1)
        def group_body(g):
            gcol = g * _CPN
            idxs = [idx_t[g * 8 + c, :] for c in range(8)]
            lpv = [lpt_v[c, pl.ds(gcol, _CPN)] for c in range(8)]
            for b in range(0, _RB, 2):
                r0 = xbuf.at[pl.ds(pl.multiple_of(xoff + b * _CH_W,
                                                  _CH_W), _CH_W)]
                r1 = xbuf.at[pl.ds(pl.multiple_of(xoff + (b + 1) * _CH_W,
                                                  _CH_W), _CH_W)]
                a0 = [None] * 2
                a1 = [None] * 2
                for c in range(8):
                    g0 = plsc.load_gather(r0, [idxs[c]])
                    g1 = plsc.load_gather(r1, [idxs[c]])
                    t0 = jnp.exp(g0 + lpv[c])
                    t1 = jnp.exp(g1 + lpv[c])
                    p0, p1 = a0[c % 2], a1[c % 2]
                    a0[c % 2] = t0 if p0 is None else p0 + t0
                    a1[c % 2] = t1 if p1 is None else p1 + t1
                stg[pl.ds(b * _SROW + gcol, _CPN)] = a0[0] + a0[1]
                stg[pl.ds((b + 1) * _SROW + gcol, _CPN)] = a1[0] + a1[1]

        @plsc.parallel_loop(0, _RB, 1, unroll=1)
        def row_body(b):
            rowref = stg.at[pl.ds(pl.multiple_of(b * _SROW, _SROW), _SROW)]
            for k in range(_NBLK_N):
                iA = patf + (k * 32)
                iB = iA + 8
                A = plsc.load_gather(rowref, [iA])
                B = plsc.load_gather(rowref, [iB])
                sl = pl.ds(k * _CPN, _CPN)
                st[soff + b, sl] = _vlog(A + B) - lse_v[sl]

    _start(0, sem0, 0)

    def blk_body(i, _):
        par = lax.rem(i, 2)
        even = par == 0

        @pl.when((i < _NBLK - 1) & even)
        def _():
            _start(1, sem1, i + 1)

        @pl.when((i < _NBLK - 1) & ~even)
        def _():
            _start(0, sem0, i + 1)

        @pl.when(even)
        def _():
            _wait(sem0)

        @pl.when(~even)
        def _():
            _wait(sem1)

        @pl.when((i > 1) & even)
        def _():
            _out_wait(osem0)

        @pl.when((i > 1) & ~even)
        def _():
            _out_wait(osem1)

        _compute(pl.multiple_of(par * _HALF, _CH_W), par * _RB)

        @pl.when(even)
        def _():
            _out_start(0, osem0, i)

        @pl.when(~even)
        def _():
            _out_start(1, osem1, i)

        return 0
    lax.fori_loop(0, _NBLK, blk_body, 0)

    _out_wait(osem0)
    _out_wait(osem1)


def kernel(input, logparams):
    mesh = plsc.VectorSubcoreMesh(core_axis_name="c", subcore_axis_name="s")
    f = pl.kernel(
        _sc_body,
        out_type=jax.ShapeDtypeStruct((_BATCH, _NUM), jnp.float32),
        mesh=mesh,
        compiler_params=pltpu.CompilerParams(needs_layout_passes=False),
        scratch_types=[
            pltpu.VMEM((_CH_W,), jnp.float32),
            pltpu.VMEM((8, _NG * _CPN), jnp.float32),
            pltpu.VMEM((_NODES_W,), jnp.float32),
            pltpu.VMEM((_NODES_W, _CPN), jnp.int32),
            pltpu.VMEM((2 * _RB * _CH_W,), jnp.float32),
            pltpu.VMEM((_RB * _SROW,), jnp.float32),
            pltpu.VMEM((2 * _RB, _NODES_W), jnp.float32),
            pltpu.SemaphoreType.DMA,
            pltpu.SemaphoreType.DMA,
            pltpu.SemaphoreType.DMA,
            pltpu.SemaphoreType.DMA,
        ],
    )
    return f(input, logparams)


# speedup vs baseline: 1.4807x; 1.0860x over previous
import jax
import jax.numpy as jnp
from jax import lax
from jax.experimental import pallas as pl
from jax.experimental.pallas import tpu as pltpu, tpu_sc as plsc

_NUM = 4096
_CHILD = 65536
_CPN = 16
_BATCH = 512
_NC, _NS = 2, 16
_NW = _NC * _NS
_CH_W = _CHILD // _NW
_NODES_W = _NUM // _NW
_NBLK_N = _NODES_W // _CPN
_NG = _NODES_W // 8
_RB = 8
_NBLK = _BATCH // _RB
_SROW = 2 * _NODES_W


def _vlog(x):
    ix = lax.bitcast_convert_type(x, jnp.int32)
    e = lax.shift_right_arithmetic(ix - 0x3F3504F3, 23)
    m = lax.bitcast_convert_type(ix - lax.shift_left(e, 23), jnp.float32)
    s = (m - 1.0) / (m + 1.0)
    z = s * s
    p = z * (jnp.float32(1.0 / 3.0) + z * (jnp.float32(0.2)
             + z * jnp.float32(1.0 / 7.0)))
    return e.astype(jnp.float32) * jnp.float32(0.6931471805599453) \
        + 2.0 * (s + s * p)


def _sc_body(x_hbm, lp_hbm, out_hbm, lp_v, lpt_v, lse_v, idx_t, xbuf,
             stg, st, sem0, sem1, osem0, osem1):
    wid = lax.axis_index("s") * _NC + lax.axis_index("c")
    ch0 = wid * _CH_W
    col0 = wid * _NODES_W
    bi = lax.iota(jnp.int32, 16)
    bi16 = bi * 16
    pat = (bi & 7) * 16 + lax.shift_left(lax.shift_right_logical(bi, 3), 3)
    patf = lax.shift_left(lax.shift_right_logical(bi, 3), 4) + (bi & 7)

    pltpu.sync_copy(lp_hbm.at[pl.ds(ch0, _CH_W)], lp_v)
    for k in range(_NBLK_N):
        acc = None
        for c in range(_CPN):
            g = plsc.load_gather(lp_v, [bi16 + (k * 256 + c)])
            e = jnp.exp(g)
            acc = e if acc is None else acc + e
        lse_v[pl.ds(k * _CPN, _CPN)] = _vlog(acc)
    for g in range(_NG):
        for c in range(8):
            idx = pat + (g * 128 + c)
            idx_t[g * 8 + c, :] = idx
            lpt_v[c, pl.ds(g * _CPN, _CPN)] = plsc.load_gather(lp_v, [idx])

    _HALF = _RB * _CH_W

    def _start(half, sem, blk):
        for b in range(_RB):
            pltpu.make_async_copy(
                x_hbm.at[blk * _RB + b, pl.ds(ch0, _CH_W)],
                xbuf.at[pl.ds(half * _HALF + b * _CH_W, _CH_W)], sem
            ).start()

    def _wait(sem):
        pltpu.make_async_copy(
            x_hbm.at[0, pl.ds(0, _HALF)], xbuf.at[pl.ds(0, _HALF)], sem
        ).wait()

    def _out_start(half, osem, blk):
        pltpu.make_async_copy(
            st.at[pl.ds(half * _RB, _RB), :],
            out_hbm.at[pl.ds(blk * _RB, _RB), pl.ds(col0, _NODES_W)],
            osem
        ).start()

    def _out_wait(osem):
        pltpu.make_async_copy(
            st.at[pl.ds(0, _RB), :],
            out_hbm.at[pl.ds(0, _RB), pl.ds(col0, _NODES_W)], osem
        ).wait()

    def _compute(xoff, soff):
        @plsc.parallel_loop(0, _NG, 1, unroll=1)
        def group_body(g):
            gcol = g * _CPN
            idxs = [idx_t[g * 8 + c, :] for c in range(8)]
            lpv = [lpt_v[c, pl.ds(gcol, _CPN)] for c in range(8)]
            for b in range(0, _RB, 2):
                r0 = xbuf.at[pl.ds(pl.multiple_of(xoff + b * _CH_W,
                                                  _CH_W), _CH_W)]
                r1 = xbuf.at[pl.ds(pl.multiple_of(xoff + (b + 1) * _CH_W,
                                                  _CH_W), _CH_W)]
                a0 = [None] * 2
                a1 = [None] * 2
                for c in range(8):
                    g0 = plsc.load_gather(r0, [idxs[c]])
                    g1 = plsc.load_gather(r1, [idxs[c]])
                    t0 = jnp.exp(g0 + lpv[c])
                    t1 = jnp.exp(g1 + lpv[c])
                    p0, p1 = a0[c % 2], a1[c % 2]
                    a0[c % 2] = t0 if p0 is None else p0 + t0
                    a1[c % 2] = t1 if p1 is None else p1 + t1
                stg[pl.ds(b * _SROW + gcol, _CPN)] = a0[0] + a0[1]
                stg[pl.ds((b + 1) * _SROW + gcol, _CPN)] = a1[0] + a1[1]

        @plsc.parallel_loop(0, _RB, 1, unroll=1)
        def row_body(b):
            rowref = stg.at[pl.ds(pl.multiple_of(b * _SROW, _SROW), _SROW)]
            for k in range(_NBLK_N):
                iA = patf + (k * 32)
                iB = iA + 8
                A = plsc.load_gather(rowref, [iA])
                B = plsc.load_gather(rowref, [iB])
                sl = pl.ds(k * _CPN, _CPN)
                st[soff + b, sl] = _vlog(A + B) - lse_v[sl]

    _start(0, sem0, 0)

    def blk_body(i, _):
        par = lax.rem(i, 2)
        even = par == 0

        @pl.when((i < _NBLK - 1) & even)
        def _():
            _start(1, sem1, i + 1)

        @pl.when((i < _NBLK - 1) & ~even)
        def _():
            _start(0, sem0, i + 1)

        @pl.when(even)
        def _():
            _wait(sem0)

        @pl.when(~even)
        def _():
            _wait(sem1)

        @pl.when((i > 1) & even)
        def _():
            _out_wait(osem0)

        @pl.when((i > 1) & ~even)
        def _():
            _out_wait(osem1)

        _compute(pl.multiple_of(par * _HALF, _CH_W), par * _RB)

        @pl.when(even)
        def _():
            _out_start(0, osem0, i)

        @pl.when(~even)
        def _():
            _out_start(1, osem1, i)

        return 0
    lax.fori_loop(0, _NBLK, blk_body, 0)

    _out_wait(osem0)
    _out_wait(osem1)


def kernel(input, logparams):
    mesh = plsc.VectorSubcoreMesh(core_axis_name="c", subcore_axis_name="s")
    f = pl.kernel(
        _sc_body,
        out_type=jax.ShapeDtypeStruct((_BATCH, _NUM), jnp.float32),
        mesh=mesh,
        compiler_params=pltpu.CompilerParams(needs_layout_passes=False),
        scratch_types=[
            pltpu.VMEM((_CH_W,), jnp.float32),
            pltpu.VMEM((8, _NG * _CPN), jnp.float32),
            pltpu.VMEM((_NODES_W,), jnp.float32),
            pltpu.VMEM((_NODES_W, _CPN), jnp.int32),
            pltpu.VMEM((2 * _RB * _CH_W,), jnp.float32),
            pltpu.VMEM((_RB * _SROW,), jnp.float32),
            pltpu.VMEM((2 * _RB, _NODES_W), jnp.float32),
            pltpu.SemaphoreType.DMA,
            pltpu.SemaphoreType.DMA,
            pltpu.SemaphoreType.DMA,
            pltpu.SemaphoreType.DMA,
        ],
    )
    return f(input, logparams)
